# Initial kernel scaffold; baseline (speedup 1.0000x reference)
import functools

import jax
import jax.numpy as jnp
from jax import lax
from jax.experimental import pallas as pl
from jax.experimental.pallas import tpu as pltpu
from jax.experimental.pallas import tpu_sc as plsc


def _make_gather(V, D, B):
    info = plsc.get_sparse_core_info()
    NC, NS = info.num_cores, info.num_subcores
    NW = NC * NS
    assert B % NW == 0
    b_per_w = B // NW
    CHUNK = 64
    assert b_per_w % CHUNK == 0
    n_chunks = b_per_w // CHUNK

    mesh = plsc.VectorSubcoreMesh(core_axis_name="c", subcore_axis_name="s")

    @functools.partial(
        pl.kernel,
        out_type=jax.ShapeDtypeStruct((B, D), jnp.float32),
        mesh=mesh,
        scratch_types=[
            pltpu.VMEM((n_chunks, CHUNK), jnp.int32),
            pltpu.VMEM((CHUNK, D), jnp.float32),
            pltpu.SemaphoreType.DMA,
        ],
    )
    def gather_k(table_hbm, idx_hbm, out_hbm, idx_v, buf_v, sem):
        wid = lax.axis_index("s") * NC + lax.axis_index("c")
        pltpu.sync_copy(idx_hbm.at[pl.ds(wid * n_chunks, n_chunks)], idx_v)
        out_base = wid * b_per_w

        def chunk_body(j, carry):
            pltpu.async_copy(table_hbm.at[idx_v.at[j]], buf_v, sem).wait()
            pltpu.sync_copy(buf_v, out_hbm.at[pl.ds(out_base + j * CHUNK, CHUNK)])
            return carry

        lax.fori_loop(0, n_chunks, chunk_body, 0)

    return gather_k


def kernel(x, table):
    BATCH, SEQ = x.shape
    V, D = table.shape
    B = BATCH * SEQ
    CHUNK = 64
    idx2d = x.reshape(B // CHUNK, CHUNK).astype(jnp.int32)
    out = _make_gather(V, D, B)(table, idx2d)
    return out.reshape(BATCH, SEQ, D)


# speedup vs baseline: 1.0135x; 1.0135x over previous
import functools

import jax
import jax.numpy as jnp
from jax import lax
from jax.experimental import pallas as pl
from jax.experimental.pallas import tpu as pltpu
from jax.experimental.pallas import tpu_sc as plsc


def _make_gather(V, D, B):
    info = plsc.get_sparse_core_info()
    NC, NS = info.num_cores, info.num_subcores
    NW = NC * NS
    assert B % NW == 0
    b_per_w = B // NW
    CHUNK = 64
    assert b_per_w % CHUNK == 0
    n_chunks = b_per_w // CHUNK

    mesh = plsc.VectorSubcoreMesh(core_axis_name="c", subcore_axis_name="s")

    @functools.partial(
        pl.kernel,
        out_type=jax.ShapeDtypeStruct((B, D), jnp.float32),
        mesh=mesh,
        scratch_types=[
            pltpu.VMEM((n_chunks, CHUNK), jnp.int32),
            pltpu.VMEM((CHUNK, D), jnp.float32),
            pltpu.SemaphoreType.DMA,
        ],
        compiler_params=pltpu.CompilerParams(use_tc_tiling_on_sc=False),
    )
    def gather_k(table_hbm, idx_hbm, out_hbm, idx_v, buf_v, sem):
        wid = lax.axis_index("s") * NC + lax.axis_index("c")
        pltpu.sync_copy(idx_hbm.at[wid], idx_v)
        out_base = wid * b_per_w

        def chunk_body(j, carry):
            pltpu.async_copy(table_hbm.at[idx_v.at[j]], buf_v, sem).wait()
            pltpu.sync_copy(buf_v, out_hbm.at[pl.ds(out_base + j * CHUNK, CHUNK)])
            return carry

        lax.fori_loop(0, n_chunks, chunk_body, 0)

    return gather_k


def kernel(x, table):
    BATCH, SEQ = x.shape
    V, D = table.shape
    B = BATCH * SEQ
    CHUNK = 64
    NW = 32
    idx3d = x.reshape(NW, (B // NW) // CHUNK, CHUNK).astype(jnp.int32)
    out = _make_gather(V, D, B)(table, idx3d)
    return out.reshape(BATCH, SEQ, D)


# speedup vs baseline: 1.0270x; 1.0133x over previous
import functools

import jax
import jax.numpy as jnp
from jax import lax
from jax.experimental import pallas as pl
from jax.experimental.pallas import tpu as pltpu
from jax.experimental.pallas import tpu_sc as plsc

_CHUNK = 50


def _make_gather(V, D, B):
    info = plsc.get_sparse_core_info()
    NC, NS = info.num_cores, info.num_subcores
    NW = NC * NS
    assert B % NW == 0
    b_per_w = B // NW
    assert b_per_w % _CHUNK == 0
    n_chunks = b_per_w // _CHUNK
    assert n_chunks % 2 == 0 and n_chunks >= 4

    mesh = plsc.VectorSubcoreMesh(core_axis_name="c", subcore_axis_name="s")

    @functools.partial(
        pl.kernel,
        out_type=jax.ShapeDtypeStruct((B, D), jnp.float32),
        mesh=mesh,
        scratch_types=[
            pltpu.VMEM((n_chunks, _CHUNK), jnp.int32),
            pltpu.VMEM((_CHUNK, D), jnp.float32),
            pltpu.VMEM((_CHUNK, D), jnp.float32),
            pltpu.SemaphoreType.DMA,
            pltpu.SemaphoreType.DMA,
            pltpu.SemaphoreType.DMA,
            pltpu.SemaphoreType.DMA,
        ],
        compiler_params=pltpu.CompilerParams(use_tc_tiling_on_sc=False),
    )
    def gather_k(table_hbm, idx_hbm, out_hbm, idx_v, buf0, buf1,
                 gsem0, gsem1, osem0, osem1):
        wid = lax.axis_index("s") * NC + lax.axis_index("c")
        pltpu.sync_copy(idx_hbm.at[wid], idx_v)
        out_base = wid * b_per_w

        bufs = (buf0, buf1)
        gsems = (gsem0, gsem1)
        osems = (osem0, osem1)

        def g_start(j, b):
            pltpu.async_copy(table_hbm.at[idx_v.at[j]], bufs[b], gsems[b])

        def g_wait(j, b):
            pltpu.make_async_copy(
                table_hbm.at[idx_v.at[j]], bufs[b], gsems[b]).wait()

        def out_ref(j):
            return out_hbm.at[pl.ds(out_base + j * _CHUNK, _CHUNK)]

        def o_start(j, b):
            pltpu.async_copy(bufs[b], out_ref(j), osems[b])

        def o_wait(j, b):
            pltpu.make_async_copy(bufs[b], out_ref(j), osems[b]).wait()

        g_start(0, 0)
        g_wait(0, 0)
        o_start(0, 0)
        g_start(1, 1)

        def pair(i, carry):
            j0 = 2 * i
            j1 = j0 + 1
            g_wait(j0 - 1, 1)
            o_wait(j0 - 2, 0)
            o_start(j0 - 1, 1)
            g_start(j0, 0)
            g_wait(j0, 0)
            o_wait(j1 - 2, 1)
            o_start(j0, 0)
            g_start(j1, 1)
            return carry

        lax.fori_loop(1, n_chunks // 2, pair, 0)

        last = n_chunks - 1
        g_wait(last, 1)
        o_wait(last - 1, 0)
        o_start(last, 1)
        o_wait(last, 1)

    return gather_k


def kernel(x, table):
    BATCH, SEQ = x.shape
    V, D = table.shape
    B = BATCH * SEQ
    NW = 32
    idx3d = x.reshape(NW, (B // NW) // _CHUNK, _CHUNK).astype(jnp.int32)
    out = _make_gather(V, D, B)(table, idx3d)
    return out.reshape(BATCH, SEQ, D)


# speedup vs baseline: 1.0287x; 1.0017x over previous
import functools

import jax
import jax.numpy as jnp
from jax import lax
from jax.experimental import pallas as pl
from jax.experimental.pallas import tpu as pltpu
from jax.experimental.pallas import tpu_sc as plsc


def _make_gather(V, D, BATCH, SEQ):
    info = plsc.get_sparse_core_info()
    NC, NS = info.num_cores, info.num_subcores
    NW = NC * NS
    assert BATCH % NW == 0
    n_chunks = BATCH // NW
    assert n_chunks % 2 == 0 and n_chunks >= 4

    mesh = plsc.VectorSubcoreMesh(core_axis_name="c", subcore_axis_name="s")

    @functools.partial(
        pl.kernel,
        out_type=jax.ShapeDtypeStruct((BATCH, SEQ, D), jnp.float32),
        mesh=mesh,
        scratch_types=[
            pltpu.VMEM((n_chunks, SEQ), jnp.int32),
            pltpu.VMEM((SEQ, D), jnp.float32),
            pltpu.VMEM((SEQ, D), jnp.float32),
            pltpu.SemaphoreType.DMA,
            pltpu.SemaphoreType.DMA,
            pltpu.SemaphoreType.DMA,
            pltpu.SemaphoreType.DMA,
        ],
        compiler_params=pltpu.CompilerParams(use_tc_tiling_on_sc=False),
    )
    def gather_k(table_hbm, idx_hbm, out_hbm, idx_v, buf0, buf1,
                 gsem0, gsem1, osem0, osem1):
        wid = lax.axis_index("s") * NC + lax.axis_index("c")
        pltpu.sync_copy(idx_hbm.at[wid], idx_v)
        out_base = wid * n_chunks

        bufs = (buf0, buf1)
        gsems = (gsem0, gsem1)
        osems = (osem0, osem1)

        def g_start(j, b):
            pltpu.async_copy(table_hbm.at[idx_v.at[j]], bufs[b], gsems[b])

        def g_wait(j, b):
            pltpu.make_async_copy(
                table_hbm.at[idx_v.at[j]], bufs[b], gsems[b]).wait()

        def o_start(j, b):
            pltpu.async_copy(bufs[b], out_hbm.at[out_base + j], osems[b])

        def o_wait(j, b):
            pltpu.make_async_copy(
                bufs[b], out_hbm.at[out_base + j], osems[b]).wait()

        g_start(0, 0)
        g_wait(0, 0)
        o_start(0, 0)
        g_start(1, 1)

        def pair(i, carry):
            j0 = 2 * i
            j1 = j0 + 1
            g_wait(j0 - 1, 1)
            o_wait(j0 - 2, 0)
            o_start(j0 - 1, 1)
            g_start(j0, 0)
            g_wait(j0, 0)
            o_wait(j1 - 2, 1)
            o_start(j0, 0)
            g_start(j1, 1)
            return carry

        lax.fori_loop(1, n_chunks // 2, pair, 0)

        last = n_chunks - 1
        g_wait(last, 1)
        o_wait(last - 1, 0)
        o_start(last, 1)
        o_wait(last, 1)

    return gather_k


def kernel(x, table):
    BATCH, SEQ = x.shape
    V, D = table.shape
    NW = 32
    idx3d = x.reshape(NW, BATCH // NW, SEQ).astype(jnp.int32)
    return _make_gather(V, D, BATCH, SEQ)(table, idx3d)
